# baseline (speedup 1.0000x reference)
import functools

import jax
import jax.numpy as jnp
from jax import lax
from jax.experimental import pallas as pl
from jax.experimental.pallas import tpu as pltpu
from jax.experimental.pallas import tpu_sc as plsc

C = 128
N = 10000
K = 32
N_PAD = 10240
NC, NS = 2, 16
NW = NC * NS
B_PER_W = N_PAD // NW
GB = 2
N_BATCH = B_PER_W // GB
NBUF = 4

TC_BLK = 512
TC_GRID = N_PAD // TC_BLK


def _h_body(x_ref, w1_ref, b1_ref, h_ref):
    h = lax.dot_general(x_ref[...], w1_ref[...],
                        dimension_numbers=(((0,), (1,)), ((), ())),
                        preferred_element_type=jnp.float32)
    h_ref[...] = jnp.maximum(h + b1_ref[...], 0.0)


def _compute_h(x_cn, w1, b1):
    return pl.pallas_call(
        _h_body,
        grid=(TC_GRID,),
        in_specs=[
            pl.BlockSpec((C, TC_BLK), lambda i: (0, i)),
            pl.BlockSpec((C, C), lambda i: (0, 0)),
            pl.BlockSpec((1, C), lambda i: (0, 0)),
        ],
        out_specs=pl.BlockSpec((TC_BLK, C), lambda i: (i, 0)),
        out_shape=jax.ShapeDtypeStruct((N_PAD, C), jnp.float32),
    )(x_cn, w1, b1.reshape(1, C))


BK = GB * K


@functools.cache
def _make_sc_gather_max():
    @functools.partial(
        pl.kernel,
        out_type=jax.ShapeDtypeStruct((N_PAD, C), jnp.float32),
        mesh=plsc.VectorSubcoreMesh(core_axis_name="c", subcore_axis_name="s"),
        scratch_types=[
            pltpu.VMEM((B_PER_W * K,), jnp.int32),
            pltpu.VMEM((NBUF, BK, C), jnp.float32),
            pltpu.VMEM((B_PER_W, C), jnp.float32),
            pltpu.SemaphoreType.DMA,
            pltpu.SemaphoreType.DMA,
            pltpu.SemaphoreType.DMA,
            pltpu.SemaphoreType.DMA,
        ],
    )
    def _sc_gather_max(table, idxs, out, idx_v, ring_v, out_v,
                       sem0, sem1, sem2, sem3):
        sems = [sem0, sem1, sem2, sem3]
        wid = lax.axis_index("s") * NC + lax.axis_index("c")
        base = wid * B_PER_W
        pltpu.sync_copy(idxs.at[pl.ds(base * K, B_PER_W * K)], idx_v)

        def start(b, r, sem):
            pltpu.async_copy(
                table.at[idx_v.at[pl.ds(b * BK, BK)]], ring_v.at[r], sem)

        def drain(r, sem):
            pltpu.make_async_copy(table.at[pl.ds(0, BK)], ring_v.at[r], sem).wait()

        def reduce_batch(b, r):
            for g in range(GB):
                for l in range(C // 16):
                    acc = ring_v[r, g * K, pl.ds(l * 16, 16)]
                    for k in range(1, K):
                        acc = jnp.maximum(
                            acc, ring_v[r, g * K + k, pl.ds(l * 16, 16)])
                    out_v[b * GB + g, pl.ds(l * 16, 16)] = acc

        for r in range(NBUF - 1):
            start(r, r, sems[r])

        def quad_body(j4, _):
            b0 = NBUF * j4
            for r in range(NBUF):
                b = b0 + r
                nb = b + NBUF - 1
                sem_n = sems[(r + NBUF - 1) % NBUF]

                @pl.when(nb < N_BATCH)
                def _():
                    start(nb, (r + NBUF - 1) % NBUF, sem_n)

                drain(r, sems[r])
                reduce_batch(b, r)
            return 0

        lax.fori_loop(0, N_BATCH // NBUF, quad_body, 0)
        pltpu.sync_copy(out_v, out.at[pl.ds(base, B_PER_W)])

    return _sc_gather_max


def _out_body(x_ref, m_ref, w2a_ref, w2b_ref, b2_ref, o_ref):
    a = lax.dot_general(w2a_ref[...], x_ref[...],
                        dimension_numbers=(((1,), (0,)), ((), ())),
                        preferred_element_type=jnp.float32)
    b = lax.dot_general(w2b_ref[...], m_ref[...],
                        dimension_numbers=(((1,), (1,)), ((), ())),
                        preferred_element_type=jnp.float32)
    o_ref[...] = jnp.maximum(a + b + b2_ref[...], 0.0)


def _compute_out(x_cn, m, w2a, w2b, b2):
    return pl.pallas_call(
        _out_body,
        grid=(TC_GRID,),
        in_specs=[
            pl.BlockSpec((C, TC_BLK), lambda i: (0, i)),
            pl.BlockSpec((TC_BLK, C), lambda i: (i, 0)),
            pl.BlockSpec((C, C), lambda i: (0, 0)),
            pl.BlockSpec((C, C), lambda i: (0, 0)),
            pl.BlockSpec((C, 1), lambda i: (0, 0)),
        ],
        out_specs=pl.BlockSpec((C, TC_BLK), lambda i: (0, i)),
        out_shape=jax.ShapeDtypeStruct((C, N_PAD), jnp.float32),
    )(x_cn, m, w2a, w2b, b2.reshape(C, 1))


def kernel(x, edge_index, W1, b1, W2, b2):
    x_cn = x[0, :, :, 0]
    x_cn = jnp.pad(x_cn, ((0, 0), (0, N_PAD - N)))
    idx = edge_index[0, 0].astype(jnp.int32)
    idx = jnp.pad(idx, ((0, N_PAD - N), (0, 0)))
    idx_flat = idx.reshape(N_PAD * K)

    h = _compute_h(x_cn, W1, b1)
    m = _make_sc_gather_max()(h, idx_flat)
    out_cn = _compute_out(x_cn, m, W2[:, :C], W2[:, C:], b2)
    return out_cn[:, :N].reshape(1, C, N, 1)


# speedup vs baseline: 1.6712x; 1.6712x over previous
import functools

import jax
import jax.numpy as jnp
from jax import lax
from jax.experimental import pallas as pl
from jax.experimental.pallas import tpu as pltpu
from jax.experimental.pallas import tpu_sc as plsc

C = 128
N = 10000
K = 32
N_PAD = 10240
NC, NS = 2, 16
NW = NC * NS
T_F = C // NW
CH_N = 512
CH_NG = CH_N // 16
N_CHUNK = N_PAD // CH_N

TC_BLK = 512
TC_GRID = N_PAD // TC_BLK


def _h_body(x_ref, w1_ref, b1_ref, h_ref):
    h = lax.dot_general(w1_ref[...], x_ref[...],
                        dimension_numbers=(((1,), (0,)), ((), ())),
                        preferred_element_type=jnp.float32)
    h_ref[...] = jnp.maximum(h + b1_ref[...], 0.0)


def _compute_h(x_cn, w1, b1):
    return pl.pallas_call(
        _h_body,
        grid=(TC_GRID,),
        in_specs=[
            pl.BlockSpec((C, TC_BLK), lambda i: (0, i)),
            pl.BlockSpec((C, C), lambda i: (0, 0)),
            pl.BlockSpec((C, 1), lambda i: (0, 0)),
        ],
        out_specs=pl.BlockSpec((C, TC_BLK), lambda i: (0, i)),
        out_shape=jax.ShapeDtypeStruct((C, N_PAD), jnp.float32),
    )(x_cn, w1, b1.reshape(C, 1))


@functools.cache
def _make_sc_gather_max():
    @functools.partial(
        pl.kernel,
        out_type=jax.ShapeDtypeStruct((NW, N_CHUNK, CH_NG, T_F, 16),
                                      jnp.float32),
        mesh=plsc.VectorSubcoreMesh(core_axis_name="c", subcore_axis_name="s"),
        compiler_params=pltpu.CompilerParams(needs_layout_passes=False,
                                             use_tc_tiling_on_sc=False),
        scratch_types=[
            pltpu.VMEM((T_F * N_PAD,), jnp.float32),
            pltpu.VMEM((2, CH_NG, K, 16), jnp.int32),
            pltpu.VMEM((2, CH_NG, T_F, 16), jnp.float32),
            pltpu.SemaphoreType.DMA,
            pltpu.SemaphoreType.DMA,
            pltpu.SemaphoreType.DMA,
            pltpu.SemaphoreType.DMA,
        ],
    )
    def _sc_gather_max(h, idx4, out, table_v, idx_ring, out_ring,
                       sem_i0, sem_i1, sem_o0, sem_o1):
        t = lax.axis_index("s") * NC + lax.axis_index("c")
        sems_i = [sem_i0, sem_i1]
        sems_o = [sem_o0, sem_o1]

        pltpu.sync_copy(h.at[pl.ds(t * (T_F * N_PAD), T_F * N_PAD)], table_v)

        def start_idx(c, rb):
            pltpu.async_copy(idx4.at[c], idx_ring.at[rb], sems_i[rb])

        def drain_idx(rb):
            pltpu.make_async_copy(idx4.at[0], idx_ring.at[rb],
                                  sems_i[rb]).wait()

        def start_out(c, rb):
            pltpu.async_copy(out_ring.at[rb], out.at[t, c], sems_o[rb])

        def drain_out(rb):
            pltpu.make_async_copy(out_ring.at[rb], out.at[t, 0],
                                  sems_o[rb]).wait()

        start_idx(0, 0)

        def chunk_pair(c2, _):
            for rb in range(2):
                c = 2 * c2 + rb

                @pl.when(c + 1 < N_CHUNK)
                def _():
                    start_idx(c + 1, 1 - rb)

                drain_idx(rb)

                @pl.when(c >= 2)
                def _():
                    drain_out(rb)

                def ng_body(ng, _):
                    ivs = [idx_ring[rb, ng, k, :] for k in range(K)]
                    for f in range(T_F):
                        fvs = [iv + (f * N_PAD) for iv in ivs] if f else ivs
                        acc = plsc.load_gather(table_v, [fvs[0]])
                        for k in range(1, K):
                            acc = jnp.maximum(
                                acc, plsc.load_gather(table_v, [fvs[k]]))
                        out_ring[rb, ng, f, :] = acc
                    return 0

                lax.fori_loop(0, CH_NG, ng_body, 0)
                start_out(c, rb)
            return 0

        lax.fori_loop(0, N_CHUNK // 2, chunk_pair, 0)
        drain_out(0)
        drain_out(1)

    return _sc_gather_max


def _out_body(x_ref, m_ref, w2a_ref, w2b_ref, b2_ref, o_ref):
    a = lax.dot_general(w2a_ref[...], x_ref[...],
                        dimension_numbers=(((1,), (0,)), ((), ())),
                        preferred_element_type=jnp.float32)
    b = lax.dot_general(w2b_ref[...], m_ref[...],
                        dimension_numbers=(((1,), (0,)), ((), ())),
                        preferred_element_type=jnp.float32)
    o_ref[...] = jnp.maximum(a + b + b2_ref[...], 0.0)


def _compute_out(x_cn, m_cn, w2a, w2b, b2):
    return pl.pallas_call(
        _out_body,
        grid=(TC_GRID,),
        in_specs=[
            pl.BlockSpec((C, TC_BLK), lambda i: (0, i)),
            pl.BlockSpec((C, TC_BLK), lambda i: (0, i)),
            pl.BlockSpec((C, C), lambda i: (0, 0)),
            pl.BlockSpec((C, C), lambda i: (0, 0)),
            pl.BlockSpec((C, 1), lambda i: (0, 0)),
        ],
        out_specs=pl.BlockSpec((C, TC_BLK), lambda i: (0, i)),
        out_shape=jax.ShapeDtypeStruct((C, N_PAD), jnp.float32),
    )(x_cn, m_cn, w2a, w2b, b2.reshape(C, 1))


def kernel(x, edge_index, W1, b1, W2, b2):
    x_cn = x[0, :, :, 0]
    x_cn = jnp.pad(x_cn, ((0, 0), (0, N_PAD - N)))
    idx = edge_index[0, 0].astype(jnp.int32)
    idx = jnp.pad(idx, ((0, N_PAD - N), (0, 0)))
    idx4 = idx.reshape(N_CHUNK, CH_NG, 16, K).transpose(0, 1, 3, 2)

    h = _compute_h(x_cn, W1, b1)
    m_raw = _make_sc_gather_max()(h.reshape(C * N_PAD), idx4)
    m_cn = m_raw.transpose(0, 3, 1, 2, 4).reshape(C, N_PAD)
    out_cn = _compute_out(x_cn, m_cn, W2[:, :C], W2[:, C:], b2)
    return out_cn[:, :N].reshape(1, C, N, 1)
